# Initial kernel scaffold; baseline (speedup 1.0000x reference)
import jax
import jax.numpy as jnp
from jax.experimental import pallas as pl


def kernel(atoms, indices):
    raise NotImplementedError("write your pallas kernel here")


# speedup vs baseline: 25.1703x; 25.1703x over previous
import functools
import math

import jax
import jax.numpy as jnp
from jax import lax
from jax.experimental import pallas as pl
from jax.experimental.pallas import tpu as pltpu
from jax.experimental.pallas import tpu_sc as plsc

BINS = 64
VMIN = 0.0
VMAX = 2.0
SIGMA = 0.05
NCLS = 32
DELTA = (VMAX - VMIN) / BINS
KNORM = DELTA / (SIGMA * math.sqrt(2.0 * math.pi))
TAPS = 16
HALF = 7

NBATCH = 4
NATOMS = 4096
NBONDS = 32768
NW = 32
BPW = NBONDS // NW
WPB = NW // NBATCH
HSZ = BINS * NCLS
L = 16


def _sc_body(atoms_hbm, i1_hbm, i2_hbm, cl_hbm, out_hbm,
             atoms_v, i1_v, i2_v, cl_v, hist_v):
    wid = lax.axis_index("s") * 2 + lax.axis_index("c")
    batch = wid // WPB
    base = wid * BPW

    pltpu.sync_copy(atoms_hbm.at[batch], atoms_v)
    pltpu.sync_copy(i1_hbm.at[pl.ds(base, BPW)], i1_v)
    pltpu.sync_copy(i2_hbm.at[pl.ds(base, BPW)], i2_v)
    pltpu.sync_copy(cl_hbm.at[pl.ds(base, BPW)], cl_v)

    zeros = jnp.zeros((L,), jnp.float32)

    def zero_body(i, carry):
        hist_v[pl.ds(i * L, L)] = zeros
        return carry

    lax.fori_loop(0, L * HSZ // L, zero_body, 0)

    lanes = jax.lax.iota(jnp.int32, L)
    lane_base = lanes * HSZ
    half_f = jnp.float32(0.5)
    inv_delta = jnp.float32(1.0 / DELTA)
    delta_f = jnp.float32(DELTA)
    inv_sigma = jnp.float32(1.0 / SIGMA)
    knorm_f = jnp.float32(KNORM)
    magic = jnp.int32(0x5F3759DF)

    def group_body(g, carry):
        off = g * L
        idx1 = i1_v[pl.ds(off, L)]
        idx2 = i2_v[pl.ds(off, L)]
        cls = cl_v[pl.ds(off, L)]

        x1 = plsc.load_gather(atoms_v, [idx1])
        y1 = plsc.load_gather(atoms_v, [idx1 + NATOMS])
        z1 = plsc.load_gather(atoms_v, [idx1 + 2 * NATOMS])
        x2 = plsc.load_gather(atoms_v, [idx2])
        y2 = plsc.load_gather(atoms_v, [idx2 + NATOMS])
        z2 = plsc.load_gather(atoms_v, [idx2 + 2 * NATOMS])

        dx = x1 - x2
        dy = y1 - y2
        dz = z1 - z2
        d2 = dx * dx + dy * dy + dz * dz

        bits = lax.bitcast_convert_type(d2, jnp.int32)
        bits = magic - lax.shift_right_arithmetic(bits, 1)
        y = lax.bitcast_convert_type(bits, jnp.float32)
        for _ in range(3):
            t = (d2 * y) * y
            y = y * (jnp.float32(1.5) - half_f * t)
        dis = d2 * y

        b0 = (dis * inv_delta).astype(jnp.int32)
        sbase = lane_base + cls

        for t in range(TAPS):
            binv = b0 + (t - HALF)
            ctr = (binv.astype(jnp.float32) + half_f) * delta_f
            u = (dis - ctr) * inv_sigma
            w = jnp.exp(-half_f * u * u) * knorm_f
            m = (binv >= 0) & (binv < BINS)
            plsc.addupdate_scatter(hist_v, [sbase + binv * NCLS], w, mask=m)
        return carry

    lax.fori_loop(0, BPW // L, group_body, 0)

    def red_body(j, carry):
        o = j * L
        acc = hist_v[pl.ds(o, L)]
        for l in range(1, L):
            acc = acc + hist_v[pl.ds(l * HSZ + o, L)]
        hist_v[pl.ds(o, L)] = acc
        return carry

    lax.fori_loop(0, HSZ // L, red_body, 0)

    pltpu.sync_copy(hist_v.at[pl.ds(0, HSZ)], out_hbm.at[wid])


@jax.jit
def _sc_hist(atoms_t, i1, i2, cl):
    mesh = plsc.VectorSubcoreMesh(core_axis_name="c", subcore_axis_name="s")
    f = functools.partial(
        pl.kernel,
        mesh=mesh,
        out_type=jax.ShapeDtypeStruct((NW, HSZ), jnp.float32),
        scratch_types=[
            pltpu.VMEM((3 * NATOMS,), jnp.float32),
            pltpu.VMEM((BPW,), jnp.int32),
            pltpu.VMEM((BPW,), jnp.int32),
            pltpu.VMEM((BPW,), jnp.int32),
            pltpu.VMEM((L * HSZ,), jnp.float32),
        ],
        compiler_params=pltpu.CompilerParams(needs_layout_passes=False),
    )(_sc_body)
    return f(atoms_t, i1, i2, cl)


def _finish_body(p_ref, o_ref):
    x = p_ref[...]
    h = jnp.sum(x, axis=0) + jnp.float32(1e-40)
    s = jnp.sum(h, axis=0, keepdims=True)
    o_ref[...] = h / (s + jnp.float32(1e-20))


@jax.jit
def _finish(partials):
    return pl.pallas_call(
        _finish_body,
        out_shape=jax.ShapeDtypeStruct((BINS, NCLS), jnp.float32),
    )(partials)


def kernel(atoms, indices):
    idx = indices.astype(jnp.int32)
    atoms_t = atoms.transpose(0, 2, 1).reshape(NBATCH, 3 * NATOMS)
    i1 = idx[:, :, 1].reshape(-1)
    i2 = idx[:, :, 2].reshape(-1)
    cl = idx[:, :, 0].reshape(-1)
    partials = _sc_hist(atoms_t, i1, i2, cl)
    res = _finish(partials.reshape(NW, BINS, NCLS))
    return res[:, : NCLS - 1]
